# baseline (speedup 1.0000x reference)
import functools

import jax
import jax.numpy as jnp
from jax import lax
from jax.experimental import pallas as pl
from jax.experimental.pallas import tpu as pltpu, tpu_sc as plsc

BUFFER_SIZE = 2000000
RESOLUTION = 0.3
NUM_BANDS = 64
FEATURE_DIM = 32
N_POINTS = 262144

NW = 32
PTS_PER_W = N_POINTS // NW
TBL = 41667
NB = 48
NG = NB // 16
CHUNK = 512
STAGE = PTS_PER_W + NB * 16
OUT_COLS = 3 + 2 * NUM_BANDS + FEATURE_DIM

P0M = 73856093 % BUFFER_SIZE
P1M = 19349669 % BUFFER_SIZE
P2M = 83492791 % BUFFER_SIZE

_SC_MESH = dict(core_axis_name="c", subcore_axis_name="s")
_i = jnp.int32
_SC_PARAMS = pltpu.CompilerParams(needs_layout_passes=False,
                                  use_tc_tiling_on_sc=False)


_KEY_BLOCK = 8192


def _keys_body(points_ref, out_ref):
    pts = points_ref[...]
    g = jnp.floor(pts / jnp.float32(RESOLUTION)).astype(jnp.int32)
    k = (g[:, 0:1] * P0M + g[:, 1:2] * P1M + g[:, 2:3] * P2M)
    out_ref[...] = jnp.mod(k, BUFFER_SIZE)


def _compute_keys(points):
    n = points.shape[0]
    return pl.pallas_call(
        _keys_body,
        grid=(n // _KEY_BLOCK,),
        in_specs=[pl.BlockSpec((_KEY_BLOCK, 3), lambda i: (i, i * 0))],
        out_specs=pl.BlockSpec((_KEY_BLOCK, 1), lambda i: (i, i * 0)),
        out_shape=jax.ShapeDtypeStruct((n, 1), jnp.int32),
    )(points)


def _sort_body(keys_hbm, pkh_hbm, lh_hbm, lo_hbm,
               d_v, pk_v, stage, lhist, loffs, wc):
    c = lax.axis_index("c")
    s = lax.axis_index("s")
    w = s * _i(2) + c
    iota = lax.iota(jnp.int32, 16)
    nvr = PTS_PER_W // 16

    pltpu.sync_copy(keys_hbm.at[pl.ds(pl.multiple_of(w * _i(PTS_PER_W), 16), PTS_PER_W)],
                    stage.at[pl.ds(0, PTS_PER_W)])

    def pre(i, _):
        kv = stage[pl.ds(i * _i(16), 16)]
        d = kv // TBL
        kl = kv - d * TBL
        d_v[pl.ds(i * _i(16), 16)] = d
        pk_v[pl.ds(i * _i(16), 16)] = kl * _i(16384) + i * _i(16) + iota
        return _i(0)

    lax.fori_loop(_i(0), _i(nvr), pre, _i(0))

    z16 = jnp.zeros((16,), jnp.int32)
    for g in range(NG):
        lhist[pl.ds(g * 16, 16)] = z16

    def hist(i, _):
        d = d_v[pl.ds(i * _i(16), 16)]
        occ, last = plsc.scan_count(d)
        plsc.addupdate_scatter(lhist, [d], occ, mask=last)
        return _i(0)

    lax.fori_loop(_i(0), _i(nvr), hist, _i(0))

    base = _i(0)
    for g in range(NG):
        cntg = lhist[pl.ds(g * 16, 16)]
        sz = ((cntg + 15) // 16) * 16
        incl = plsc.cumsum(sz)
        loffs[pl.ds(g * 16, 16)] = incl - sz + base
        wc[pl.ds(g * 16, 16)] = incl - sz + base
        base = base + jnp.sum(sz, dtype=jnp.int32)

    def perm(i, _):
        d = d_v[pl.ds(i * _i(16), 16)]
        pk = pk_v[pl.ds(i * _i(16), 16)]
        occ, last = plsc.scan_count(d)
        cur = plsc.load_gather(wc, [d])
        pos = cur + occ - 1
        plsc.store_scatter(wc, [d], pos + 1, mask=last)
        plsc.store_scatter(stage, [pos], pk)
        return _i(0)

    lax.fori_loop(_i(0), _i(nvr), perm, _i(0))

    pltpu.sync_copy(stage.at[pl.ds(0, STAGE)],
                    pkh_hbm.at[pl.ds(pl.multiple_of(w * _i(STAGE), 16), STAGE)])
    pltpu.sync_copy(lhist, lh_hbm.at[w])
    pltpu.sync_copy(loffs, lo_hbm.at[w])


def _sc_sort(keys):
    return pl.kernel(
        _sort_body,
        out_type=[jax.ShapeDtypeStruct((NW * STAGE + CHUNK,), jnp.int32),
                  jax.ShapeDtypeStruct((NW, NB), jnp.int32),
                  jax.ShapeDtypeStruct((NW, NB), jnp.int32)],
        mesh=plsc.VectorSubcoreMesh(**_SC_MESH),
        scratch_types=[pltpu.VMEM((PTS_PER_W,), jnp.int32),
                       pltpu.VMEM((PTS_PER_W,), jnp.int32),
                       pltpu.VMEM((STAGE,), jnp.int32),
                       pltpu.VMEM((NB,), jnp.int32),
                       pltpu.VMEM((NB,), jnp.int32),
                       pltpu.VMEM((NB,), jnp.int32)],
        compiler_params=_SC_PARAMS,
    )(keys)


def _acc_body(pkh_hbm, lh_hbm, lo_hbm, values_hbm, gout_hbm,
              table, lh_v, lo_v, pkbuf, kl_buf, idx_buf, vrows, sem_c):
    c = lax.axis_index("c")
    s = lax.axis_index("s")
    iota = lax.iota(jnp.int32, 16)
    pltpu.sync_copy(lh_hbm, lh_v)
    pltpu.sync_copy(lo_hbm, lo_v)

    def zv(r, _):
        vrows[r, pl.ds(0, 16)] = jnp.zeros((16,), jnp.float32)
        vrows[r, pl.ds(16, 16)] = jnp.zeros((16,), jnp.float32)
        return _i(0)

    lax.fori_loop(_i(0), _i(128), zv, _i(0))

    def run_stage(phase, u, m, seg):
        if phase == "out":
            dummy_idx = _i(N_POINTS) + iota
        else:
            dummy_idx = iota * _i(32) + s

        def chunk_body(k, _):
            off = pl.multiple_of(u * _i(STAGE) + seg + k * _i(CHUNK), 16)
            pltpu.sync_copy(pkh_hbm.at[pl.ds(off, CHUNK)], pkbuf)
            for j in range(CHUNK // 16):
                r = j // 8
                col = (j % 8) * 16
                lanepos = k * _i(CHUNK) + _i(j * 16) + iota
                valid = lanepos < m
                pk = pkbuf[pl.ds(_i(j * 16), 16)]
                kl = pk // _i(16384)
                pos = pk - kl * _i(16384)
                kl_buf[r, pl.ds(col, 16)] = jnp.where(
                    valid, kl, _i(TBL) + iota)
                idx_buf[r, pl.ds(col, 16)] = jnp.where(
                    valid, u * _i(PTS_PER_W) + pos, dummy_idx)
            ngr = CHUNK // 128
            if phase == "zero":
                cps = [pltpu.async_copy(
                    vrows.at[pl.ds(0, 128)],
                    table.at[kl_buf.at[_i(g4)]], sem_c) for g4 in range(ngr)]
                for cp in cps:
                    cp.wait()
            elif phase == "add":
                cps = [pltpu.async_copy(
                    values_hbm.at[idx_buf.at[_i(g4)]],
                    vrows.at[pl.ds(g4 * 128, 128)], sem_c)
                    for g4 in range(ngr)]
                for cp in cps:
                    cp.wait()
                cps = [pltpu.async_copy(
                    vrows.at[pl.ds(g4 * 128, 128)],
                    table.at[kl_buf.at[_i(g4)]], sem_c, add=True)
                    for g4 in range(ngr)]
                for cp in cps:
                    cp.wait()
            else:
                cps = [pltpu.async_copy(
                    table.at[kl_buf.at[_i(g4)]],
                    vrows.at[pl.ds(g4 * 128, 128)], sem_c)
                    for g4 in range(ngr)]
                for cp in cps:
                    cp.wait()
                cps = [pltpu.async_copy(
                    vrows.at[pl.ds(g4 * 128, 128)],
                    gout_hbm.at[idx_buf.at[_i(g4)]], sem_c)
                    for g4 in range(ngr)]
                for cp in cps:
                    cp.wait()
            return _i(0)

        nch = (m + _i(CHUNK - 1)) // _i(CHUNK)
        lax.fori_loop(_i(0), nch, chunk_body, _i(0))

    def pass_body(p, _):
        b = p * _i(2) + c
        g16 = (b // _i(16)) * _i(16)
        lane = b % _i(16)
        sel = (iota == lane)

        def do_phase(phase):
            for ui in range(2):
                u = s * _i(2) + _i(ui)
                m = jnp.sum(jnp.where(sel, lh_v[u, pl.ds(g16, 16)], _i(0)),
                            dtype=jnp.int32)
                seg = jnp.sum(jnp.where(sel, lo_v[u, pl.ds(g16, 16)], _i(0)),
                              dtype=jnp.int32)
                run_stage(phase, u, m, seg)

        do_phase("zero")
        plsc.subcore_barrier()
        do_phase("add")
        plsc.subcore_barrier()
        do_phase("out")
        plsc.subcore_barrier()
        lax.fori_loop(_i(0), _i(128), zv, _i(0))
        return _i(0)

    lax.fori_loop(_i(0), _i(NB // 2), pass_body, _i(0))


def _sc_accumulate(pkh, lh, lo, values):
    return pl.kernel(
        _acc_body,
        out_type=jax.ShapeDtypeStruct((N_POINTS + 16, FEATURE_DIM),
                                      jnp.float32),
        mesh=plsc.VectorSubcoreMesh(**_SC_MESH),
        scratch_types=[pltpu.VMEM_SHARED((TBL + 24, FEATURE_DIM),
                                         jnp.float32),
                       pltpu.VMEM((NW, NB), jnp.int32),
                       pltpu.VMEM((NW, NB), jnp.int32),
                       pltpu.VMEM((CHUNK,), jnp.int32),
                       pltpu.VMEM((CHUNK // 128, 128), jnp.int32),
                       pltpu.VMEM((CHUNK // 128, 128), jnp.int32),
                       pltpu.VMEM((CHUNK, FEATURE_DIM), jnp.float32),
                       pltpu.SemaphoreType.DMA],
        compiler_params=_SC_PARAMS,
    )(pkh, lh, lo, values)


_PE_BLOCK = 2048


def _pe_body(points_ref, bpe_ref, gathered_ref, out_ref):
    pts = points_ref[...]
    bpe = bpe_ref[...]
    px = pts[:, 0:1]
    py = pts[:, 1:2]
    pz = pts[:, 2:3]
    two_pi = 2.0 * jnp.pi
    bf = lambda a: a.astype(jnp.bfloat16).astype(jnp.float32)
    xp = (bf(px) * bf(bpe[0:1, :]) + bf(py) * bf(bpe[1:2, :])
          + bf(pz) * bf(bpe[2:3, :])) * two_pi
    c1 = jnp.float32(6.28125)
    c2 = jnp.float32(0.0019350052)
    c3 = jnp.float32(3.0198134e-07)
    c4 = jnp.float32(1.0253132e-11)
    n = jnp.round(xp * jnp.float32(1.0 / two_pi))
    r = (((xp - n * c1) - n * c2) - n * c3) - n * c4
    out_ref[...] = jnp.concatenate(
        [pts, jnp.sin(r), jnp.cos(r), gathered_ref[...]], axis=1)


def _pe_concat(points, B_pe, gathered):
    n = points.shape[0]
    return pl.pallas_call(
        _pe_body,
        grid=(n // _PE_BLOCK,),
        in_specs=[
            pl.BlockSpec((_PE_BLOCK, 3), lambda i: (i, i * 0)),
            pl.BlockSpec((3, NUM_BANDS), lambda i: (i * 0, i * 0)),
            pl.BlockSpec((_PE_BLOCK, FEATURE_DIM), lambda i: (i, i * 0)),
        ],
        out_specs=pl.BlockSpec((_PE_BLOCK, OUT_COLS), lambda i: (i, i * 0)),
        out_shape=jax.ShapeDtypeStruct((n, OUT_COLS), jnp.float32),
    )(points, B_pe, gathered)


def kernel(points, values, mem, B_pe):
    del mem
    keys = _compute_keys(points).reshape((N_POINTS,))
    pkh, lh, lo = _sc_sort(keys)
    gout = _sc_accumulate(pkh, lh, lo, values)
    return _pe_concat(points, B_pe, gout)


# speedup vs baseline: 1.6555x; 1.6555x over previous
import functools

import jax
import jax.numpy as jnp
from jax import lax
from jax.experimental import pallas as pl
from jax.experimental.pallas import tpu as pltpu, tpu_sc as plsc

BUFFER_SIZE = 2000000
RESOLUTION = 0.3
NUM_BANDS = 64
FEATURE_DIM = 32
N_POINTS = 262144

NW = 32
PTS_PER_W = N_POINTS // NW
TBL = 41667
NB = 48
NG = NB // 16
CHUNK = 512
STAGE = PTS_PER_W + NB * 16
OUT_COLS = 3 + 2 * NUM_BANDS + FEATURE_DIM

P0M = 73856093 % BUFFER_SIZE
P1M = 19349669 % BUFFER_SIZE
P2M = 83492791 % BUFFER_SIZE

_SC_MESH = dict(core_axis_name="c", subcore_axis_name="s")
_i = jnp.int32
_SC_PARAMS = pltpu.CompilerParams(needs_layout_passes=False,
                                  use_tc_tiling_on_sc=False)


_KEY_BLOCK = 8192


def _keys_body(points_ref, out_ref):
    pts = points_ref[...]
    g = jnp.floor(pts / jnp.float32(RESOLUTION)).astype(jnp.int32)
    k = (g[:, 0:1] * P0M + g[:, 1:2] * P1M + g[:, 2:3] * P2M)
    out_ref[...] = jnp.mod(k, BUFFER_SIZE)


def _compute_keys(points):
    n = points.shape[0]
    return pl.pallas_call(
        _keys_body,
        grid=(n // _KEY_BLOCK,),
        in_specs=[pl.BlockSpec((_KEY_BLOCK, 3), lambda i: (i, i * 0))],
        out_specs=pl.BlockSpec((_KEY_BLOCK, 1), lambda i: (i, i * 0)),
        out_shape=jax.ShapeDtypeStruct((n, 1), jnp.int32),
    )(points)


def _sort_body(keys_hbm, pkh_hbm, lh_hbm, lo_hbm,
               d_v, pk_v, stage, lhist, loffs, wc):
    c = lax.axis_index("c")
    s = lax.axis_index("s")
    w = s * _i(2) + c
    iota = lax.iota(jnp.int32, 16)
    nvr = PTS_PER_W // 16

    pltpu.sync_copy(keys_hbm.at[pl.ds(pl.multiple_of(w * _i(PTS_PER_W), 16), PTS_PER_W)],
                    stage.at[pl.ds(0, PTS_PER_W)])

    def pre(i, _):
        kv = stage[pl.ds(i * _i(16), 16)]
        d = kv // TBL
        kl = kv - d * TBL
        d_v[pl.ds(i * _i(16), 16)] = d
        pk_v[pl.ds(i * _i(16), 16)] = kl * _i(16384) + i * _i(16) + iota
        return _i(0)

    lax.fori_loop(_i(0), _i(nvr), pre, _i(0))

    z16 = jnp.zeros((16,), jnp.int32)
    for g in range(NG):
        lhist[pl.ds(g * 16, 16)] = z16

    def hist(i, _):
        d = d_v[pl.ds(i * _i(16), 16)]
        occ, last = plsc.scan_count(d)
        plsc.addupdate_scatter(lhist, [d], occ, mask=last)
        return _i(0)

    lax.fori_loop(_i(0), _i(nvr), hist, _i(0))

    base = _i(0)
    for g in range(NG):
        cntg = lhist[pl.ds(g * 16, 16)]
        sz = ((cntg + 15) // 16) * 16
        incl = plsc.cumsum(sz)
        loffs[pl.ds(g * 16, 16)] = incl - sz + base
        wc[pl.ds(g * 16, 16)] = incl - sz + base
        base = base + jnp.sum(sz, dtype=jnp.int32)

    def perm(i, _):
        d = d_v[pl.ds(i * _i(16), 16)]
        pk = pk_v[pl.ds(i * _i(16), 16)]
        occ, last = plsc.scan_count(d)
        cur = plsc.load_gather(wc, [d])
        pos = cur + occ - 1
        plsc.store_scatter(wc, [d], pos + 1, mask=last)
        plsc.store_scatter(stage, [pos], pk)
        return _i(0)

    lax.fori_loop(_i(0), _i(nvr), perm, _i(0))

    pltpu.sync_copy(stage.at[pl.ds(0, STAGE)],
                    pkh_hbm.at[pl.ds(pl.multiple_of(w * _i(STAGE), 16), STAGE)])
    pltpu.sync_copy(lhist, lh_hbm.at[w])
    pltpu.sync_copy(loffs, lo_hbm.at[w])


def _sc_sort(keys):
    return pl.kernel(
        _sort_body,
        out_type=[jax.ShapeDtypeStruct((NW * STAGE + CHUNK,), jnp.int32),
                  jax.ShapeDtypeStruct((NW, NB), jnp.int32),
                  jax.ShapeDtypeStruct((NW, NB), jnp.int32)],
        mesh=plsc.VectorSubcoreMesh(**_SC_MESH),
        scratch_types=[pltpu.VMEM((PTS_PER_W,), jnp.int32),
                       pltpu.VMEM((PTS_PER_W,), jnp.int32),
                       pltpu.VMEM((STAGE,), jnp.int32),
                       pltpu.VMEM((NB,), jnp.int32),
                       pltpu.VMEM((NB,), jnp.int32),
                       pltpu.VMEM((NB,), jnp.int32)],
        compiler_params=_SC_PARAMS,
    )(keys)


def _acc_body(pkh_hbm, lh_hbm, lo_hbm, values_hbm, gout_hbm,
              table, lh_v, lo_v, pkbuf, kl_buf, idx_buf, vrows, sem_c):
    c = lax.axis_index("c")
    s = lax.axis_index("s")
    iota = lax.iota(jnp.int32, 16)
    pltpu.sync_copy(lh_hbm, lh_v)
    pltpu.sync_copy(lo_hbm, lo_v)

    def zv(r, _):
        vrows[r, pl.ds(0, 16)] = jnp.zeros((16,), jnp.float32)
        vrows[r, pl.ds(16, 16)] = jnp.zeros((16,), jnp.float32)
        return _i(0)

    lax.fori_loop(_i(0), _i(128), zv, _i(0))

    def run_stage(phase, u, m, seg):
        if phase == "out":
            dummy_idx = _i(N_POINTS) + iota
        else:
            dummy_idx = iota * _i(32) + s

        def chunk_body(k, _):
            off = pl.multiple_of(u * _i(STAGE) + seg + k * _i(CHUNK), 16)
            pltpu.sync_copy(pkh_hbm.at[pl.ds(off, CHUNK)], pkbuf)
            for j in range(CHUNK // 16):
                r = j // 8
                col = (j % 8) * 16
                lanepos = k * _i(CHUNK) + _i(j * 16) + iota
                valid = lanepos < m
                pk = pkbuf[pl.ds(_i(j * 16), 16)]
                kl = pk // _i(16384)
                pos = pk - kl * _i(16384)
                kl_buf[r, pl.ds(col, 16)] = jnp.where(
                    valid, kl, _i(TBL) + iota)
                idx_buf[r, pl.ds(col, 16)] = jnp.where(
                    valid, u * _i(PTS_PER_W) + pos, dummy_idx)
            ngr = CHUNK // 128
            mv = m - k * _i(CHUNK)
            for g4 in range(ngr):
                @pl.when(_i(g4 * 128) < mv)
                def _(g4=g4):
                    if phase == "zero":
                        pltpu.async_copy(
                            vrows.at[pl.ds(0, 128)],
                            table.at[kl_buf.at[_i(g4)]], sem_c).wait()
                    elif phase == "add":
                        pltpu.async_copy(
                            values_hbm.at[idx_buf.at[_i(g4)]],
                            vrows.at[pl.ds(g4 * 128, 128)], sem_c).wait()
                        pltpu.async_copy(
                            vrows.at[pl.ds(g4 * 128, 128)],
                            table.at[kl_buf.at[_i(g4)]], sem_c,
                            add=True).wait()
                    else:
                        pltpu.async_copy(
                            table.at[kl_buf.at[_i(g4)]],
                            vrows.at[pl.ds(g4 * 128, 128)], sem_c).wait()
                        pltpu.async_copy(
                            vrows.at[pl.ds(g4 * 128, 128)],
                            gout_hbm.at[idx_buf.at[_i(g4)]], sem_c).wait()
            return _i(0)

        nch = (m + _i(CHUNK - 1)) // _i(CHUNK)
        lax.fori_loop(_i(0), nch, chunk_body, _i(0))

    def pass_body(p, _):
        b = p * _i(2) + c
        g16 = (b // _i(16)) * _i(16)
        lane = b % _i(16)
        sel = (iota == lane)

        def do_phase(phase):
            for ui in range(2):
                u = s * _i(2) + _i(ui)
                m = jnp.sum(jnp.where(sel, lh_v[u, pl.ds(g16, 16)], _i(0)),
                            dtype=jnp.int32)
                seg = jnp.sum(jnp.where(sel, lo_v[u, pl.ds(g16, 16)], _i(0)),
                              dtype=jnp.int32)
                run_stage(phase, u, m, seg)

        do_phase("zero")
        plsc.subcore_barrier()
        do_phase("add")
        plsc.subcore_barrier()
        do_phase("out")
        plsc.subcore_barrier()
        lax.fori_loop(_i(0), _i(128), zv, _i(0))
        return _i(0)

    lax.fori_loop(_i(0), _i(NB // 2), pass_body, _i(0))


def _sc_accumulate(pkh, lh, lo, values):
    return pl.kernel(
        _acc_body,
        out_type=jax.ShapeDtypeStruct((N_POINTS + 16, FEATURE_DIM),
                                      jnp.float32),
        mesh=plsc.VectorSubcoreMesh(**_SC_MESH),
        scratch_types=[pltpu.VMEM_SHARED((TBL + 24, FEATURE_DIM),
                                         jnp.float32),
                       pltpu.VMEM((NW, NB), jnp.int32),
                       pltpu.VMEM((NW, NB), jnp.int32),
                       pltpu.VMEM((CHUNK,), jnp.int32),
                       pltpu.VMEM((CHUNK // 128, 128), jnp.int32),
                       pltpu.VMEM((CHUNK // 128, 128), jnp.int32),
                       pltpu.VMEM((CHUNK, FEATURE_DIM), jnp.float32),
                       pltpu.SemaphoreType.DMA],
        compiler_params=_SC_PARAMS,
    )(pkh, lh, lo, values)


_PE_BLOCK = 2048


def _pe_body(points_ref, bpe_ref, gathered_ref, out_ref):
    pts = points_ref[...]
    bpe = bpe_ref[...]
    px = pts[:, 0:1]
    py = pts[:, 1:2]
    pz = pts[:, 2:3]
    two_pi = 2.0 * jnp.pi
    bf = lambda a: a.astype(jnp.bfloat16).astype(jnp.float32)
    xp = (bf(px) * bf(bpe[0:1, :]) + bf(py) * bf(bpe[1:2, :])
          + bf(pz) * bf(bpe[2:3, :])) * two_pi
    c1 = jnp.float32(6.28125)
    c2 = jnp.float32(0.0019350052)
    c3 = jnp.float32(3.0198134e-07)
    c4 = jnp.float32(1.0253132e-11)
    n = jnp.round(xp * jnp.float32(1.0 / two_pi))
    r = (((xp - n * c1) - n * c2) - n * c3) - n * c4
    out_ref[...] = jnp.concatenate(
        [pts, jnp.sin(r), jnp.cos(r), gathered_ref[...]], axis=1)


def _pe_concat(points, B_pe, gathered):
    n = points.shape[0]
    return pl.pallas_call(
        _pe_body,
        grid=(n // _PE_BLOCK,),
        in_specs=[
            pl.BlockSpec((_PE_BLOCK, 3), lambda i: (i, i * 0)),
            pl.BlockSpec((3, NUM_BANDS), lambda i: (i * 0, i * 0)),
            pl.BlockSpec((_PE_BLOCK, FEATURE_DIM), lambda i: (i, i * 0)),
        ],
        out_specs=pl.BlockSpec((_PE_BLOCK, OUT_COLS), lambda i: (i, i * 0)),
        out_shape=jax.ShapeDtypeStruct((n, OUT_COLS), jnp.float32),
    )(points, B_pe, gathered)


def kernel(points, values, mem, B_pe):
    del mem
    keys = _compute_keys(points).reshape((N_POINTS,))
    pkh, lh, lo = _sc_sort(keys)
    gout = _sc_accumulate(pkh, lh, lo, values)
    return _pe_concat(points, B_pe, gout)


# speedup vs baseline: 1.6668x; 1.0068x over previous
import functools

import jax
import jax.numpy as jnp
from jax import lax
from jax.experimental import pallas as pl
from jax.experimental.pallas import tpu as pltpu, tpu_sc as plsc

BUFFER_SIZE = 2000000
RESOLUTION = 0.3
NUM_BANDS = 64
FEATURE_DIM = 32
N_POINTS = 262144

NW = 32
PTS_PER_W = N_POINTS // NW
TBL = 41667
NB = 48
NG = NB // 16
CHUNK = 512
STAGE = PTS_PER_W + NB * 16
OUT_COLS = 3 + 2 * NUM_BANDS + FEATURE_DIM

P0M = 73856093 % BUFFER_SIZE
P1M = 19349669 % BUFFER_SIZE
P2M = 83492791 % BUFFER_SIZE

_SC_MESH = dict(core_axis_name="c", subcore_axis_name="s")
_i = jnp.int32
_SC_PARAMS = pltpu.CompilerParams(needs_layout_passes=False,
                                  use_tc_tiling_on_sc=False)


_KEY_BLOCK = 8192


def _keys_body(points_ref, out_ref):
    pts = points_ref[...]
    g = jnp.floor(pts / jnp.float32(RESOLUTION)).astype(jnp.int32)
    k = (g[:, 0:1] * P0M + g[:, 1:2] * P1M + g[:, 2:3] * P2M)
    out_ref[...] = jnp.mod(k, BUFFER_SIZE)


def _compute_keys(points):
    n = points.shape[0]
    return pl.pallas_call(
        _keys_body,
        grid=(n // _KEY_BLOCK,),
        in_specs=[pl.BlockSpec((_KEY_BLOCK, 3), lambda i: (i, i * 0))],
        out_specs=pl.BlockSpec((_KEY_BLOCK, 1), lambda i: (i, i * 0)),
        out_shape=jax.ShapeDtypeStruct((n, 1), jnp.int32),
    )(points)


def _sort_body(keys_hbm, pkh_hbm, lh_hbm, lo_hbm,
               d_v, pk_v, stage, lhist, loffs, wc):
    c = lax.axis_index("c")
    s = lax.axis_index("s")
    w = s * _i(2) + c
    iota = lax.iota(jnp.int32, 16)
    nvr = PTS_PER_W // 16

    pltpu.sync_copy(keys_hbm.at[pl.ds(pl.multiple_of(w * _i(PTS_PER_W), 16), PTS_PER_W)],
                    stage.at[pl.ds(0, PTS_PER_W)])

    def pre(i, _):
        kv = stage[pl.ds(i * _i(16), 16)]
        d = kv // TBL
        kl = kv - d * TBL
        d_v[pl.ds(i * _i(16), 16)] = d
        pk_v[pl.ds(i * _i(16), 16)] = kl * _i(16384) + i * _i(16) + iota
        return _i(0)

    lax.fori_loop(_i(0), _i(nvr), pre, _i(0))

    z16 = jnp.zeros((16,), jnp.int32)
    for g in range(NG):
        lhist[pl.ds(g * 16, 16)] = z16

    def hist(i, _):
        d = d_v[pl.ds(i * _i(16), 16)]
        occ, last = plsc.scan_count(d)
        plsc.addupdate_scatter(lhist, [d], occ, mask=last)
        return _i(0)

    lax.fori_loop(_i(0), _i(nvr), hist, _i(0))

    base = _i(0)
    for g in range(NG):
        cntg = lhist[pl.ds(g * 16, 16)]
        sz = ((cntg + 15) // 16) * 16
        incl = plsc.cumsum(sz)
        loffs[pl.ds(g * 16, 16)] = incl - sz + base
        wc[pl.ds(g * 16, 16)] = incl - sz + base
        base = base + jnp.sum(sz, dtype=jnp.int32)

    def perm(i, _):
        d = d_v[pl.ds(i * _i(16), 16)]
        pk = pk_v[pl.ds(i * _i(16), 16)]
        occ, last = plsc.scan_count(d)
        cur = plsc.load_gather(wc, [d])
        pos = cur + occ - 1
        plsc.store_scatter(wc, [d], pos + 1, mask=last)
        plsc.store_scatter(stage, [pos], pk)
        return _i(0)

    lax.fori_loop(_i(0), _i(nvr), perm, _i(0))

    pltpu.sync_copy(stage.at[pl.ds(0, STAGE)],
                    pkh_hbm.at[pl.ds(pl.multiple_of(w * _i(STAGE), 16), STAGE)])
    pltpu.sync_copy(lhist, lh_hbm.at[w])
    pltpu.sync_copy(loffs, lo_hbm.at[w])


def _sc_sort(keys):
    return pl.kernel(
        _sort_body,
        out_type=[jax.ShapeDtypeStruct((NW * STAGE + CHUNK,), jnp.int32),
                  jax.ShapeDtypeStruct((NW, NB), jnp.int32),
                  jax.ShapeDtypeStruct((NW, NB), jnp.int32)],
        mesh=plsc.VectorSubcoreMesh(**_SC_MESH),
        scratch_types=[pltpu.VMEM((PTS_PER_W,), jnp.int32),
                       pltpu.VMEM((PTS_PER_W,), jnp.int32),
                       pltpu.VMEM((STAGE,), jnp.int32),
                       pltpu.VMEM((NB,), jnp.int32),
                       pltpu.VMEM((NB,), jnp.int32),
                       pltpu.VMEM((NB,), jnp.int32)],
        compiler_params=_SC_PARAMS,
    )(keys)


def _acc_body(pkh_hbm, lh_hbm, lo_hbm, values_hbm, gout_hbm,
              table, lh_v, lo_v, pkbuf, kl_buf, idx_buf, vrows, sem_c):
    c = lax.axis_index("c")
    s = lax.axis_index("s")
    iota = lax.iota(jnp.int32, 16)
    pltpu.sync_copy(lh_hbm, lh_v)
    pltpu.sync_copy(lo_hbm, lo_v)

    def zv(r, _):
        vrows[r, pl.ds(0, 16)] = jnp.zeros((16,), jnp.float32)
        vrows[r, pl.ds(16, 16)] = jnp.zeros((16,), jnp.float32)
        return _i(0)

    lax.fori_loop(_i(0), _i(128), zv, _i(0))

    def run_stage(phase, u, m, seg):
        if phase == "out":
            dummy_idx = _i(N_POINTS) + iota
        else:
            dummy_idx = iota * _i(32) + s

        def chunk_body(k, _):
            off = pl.multiple_of(u * _i(STAGE) + seg + k * _i(CHUNK), 16)
            pltpu.sync_copy(pkh_hbm.at[pl.ds(off, CHUNK)], pkbuf)
            for j in range(CHUNK // 16):
                r = j // 8
                col = (j % 8) * 16
                lanepos = k * _i(CHUNK) + _i(j * 16) + iota
                valid = lanepos < m
                pk = pkbuf[pl.ds(_i(j * 16), 16)]
                kl = pk // _i(16384)
                pos = pk - kl * _i(16384)
                kl_buf[r, pl.ds(col, 16)] = jnp.where(
                    valid, kl, _i(TBL) + iota)
                idx_buf[r, pl.ds(col, 16)] = jnp.where(
                    valid, u * _i(PTS_PER_W) + pos, dummy_idx)
            ngr = CHUNK // 128
            mv = m - k * _i(CHUNK)

            def pred_all(mk, add=False):
                cps = [mk(g4) for g4 in range(ngr)]
                for g4, cp in enumerate(cps):
                    @pl.when(_i(g4 * 128) < mv)
                    def _(cp=cp):
                        cp.start(add=add)
                for g4, cp in enumerate(cps):
                    @pl.when(_i(g4 * 128) < mv)
                    def _(cp=cp):
                        cp.wait()

            if phase == "zero":
                pred_all(lambda g4: pltpu.make_async_copy(
                    vrows.at[pl.ds(0, 128)],
                    table.at[kl_buf.at[_i(g4)]], sem_c))
            elif phase == "add":
                pred_all(lambda g4: pltpu.make_async_copy(
                    values_hbm.at[idx_buf.at[_i(g4)]],
                    vrows.at[pl.ds(g4 * 128, 128)], sem_c))
                pred_all(lambda g4: pltpu.make_async_copy(
                    vrows.at[pl.ds(g4 * 128, 128)],
                    table.at[kl_buf.at[_i(g4)]], sem_c), add=True)
            else:
                pred_all(lambda g4: pltpu.make_async_copy(
                    table.at[kl_buf.at[_i(g4)]],
                    vrows.at[pl.ds(g4 * 128, 128)], sem_c))
                pred_all(lambda g4: pltpu.make_async_copy(
                    vrows.at[pl.ds(g4 * 128, 128)],
                    gout_hbm.at[idx_buf.at[_i(g4)]], sem_c))
            return _i(0)

        nch = (m + _i(CHUNK - 1)) // _i(CHUNK)
        lax.fori_loop(_i(0), nch, chunk_body, _i(0))

    def pass_body(p, _):
        b = p * _i(2) + c
        g16 = (b // _i(16)) * _i(16)
        lane = b % _i(16)
        sel = (iota == lane)

        def do_phase(phase):
            for ui in range(2):
                u = s * _i(2) + _i(ui)
                m = jnp.sum(jnp.where(sel, lh_v[u, pl.ds(g16, 16)], _i(0)),
                            dtype=jnp.int32)
                seg = jnp.sum(jnp.where(sel, lo_v[u, pl.ds(g16, 16)], _i(0)),
                              dtype=jnp.int32)
                run_stage(phase, u, m, seg)

        do_phase("zero")
        plsc.subcore_barrier()
        do_phase("add")
        plsc.subcore_barrier()
        do_phase("out")
        plsc.subcore_barrier()
        lax.fori_loop(_i(0), _i(128), zv, _i(0))
        return _i(0)

    lax.fori_loop(_i(0), _i(NB // 2), pass_body, _i(0))


def _sc_accumulate(pkh, lh, lo, values):
    return pl.kernel(
        _acc_body,
        out_type=jax.ShapeDtypeStruct((N_POINTS + 16, FEATURE_DIM),
                                      jnp.float32),
        mesh=plsc.VectorSubcoreMesh(**_SC_MESH),
        scratch_types=[pltpu.VMEM_SHARED((TBL + 24, FEATURE_DIM),
                                         jnp.float32),
                       pltpu.VMEM((NW, NB), jnp.int32),
                       pltpu.VMEM((NW, NB), jnp.int32),
                       pltpu.VMEM((CHUNK,), jnp.int32),
                       pltpu.VMEM((CHUNK // 128, 128), jnp.int32),
                       pltpu.VMEM((CHUNK // 128, 128), jnp.int32),
                       pltpu.VMEM((CHUNK, FEATURE_DIM), jnp.float32),
                       pltpu.SemaphoreType.DMA],
        compiler_params=_SC_PARAMS,
    )(pkh, lh, lo, values)


_PE_BLOCK = 2048


def _pe_body(points_ref, bpe_ref, gathered_ref, out_ref):
    pts = points_ref[...]
    bpe = bpe_ref[...]
    px = pts[:, 0:1]
    py = pts[:, 1:2]
    pz = pts[:, 2:3]
    two_pi = 2.0 * jnp.pi
    bf = lambda a: a.astype(jnp.bfloat16).astype(jnp.float32)
    xp = (bf(px) * bf(bpe[0:1, :]) + bf(py) * bf(bpe[1:2, :])
          + bf(pz) * bf(bpe[2:3, :])) * two_pi
    c1 = jnp.float32(6.28125)
    c2 = jnp.float32(0.0019350052)
    c3 = jnp.float32(3.0198134e-07)
    c4 = jnp.float32(1.0253132e-11)
    n = jnp.round(xp * jnp.float32(1.0 / two_pi))
    r = (((xp - n * c1) - n * c2) - n * c3) - n * c4
    out_ref[...] = jnp.concatenate(
        [pts, jnp.sin(r), jnp.cos(r), gathered_ref[...]], axis=1)


def _pe_concat(points, B_pe, gathered):
    n = points.shape[0]
    return pl.pallas_call(
        _pe_body,
        grid=(n // _PE_BLOCK,),
        in_specs=[
            pl.BlockSpec((_PE_BLOCK, 3), lambda i: (i, i * 0)),
            pl.BlockSpec((3, NUM_BANDS), lambda i: (i * 0, i * 0)),
            pl.BlockSpec((_PE_BLOCK, FEATURE_DIM), lambda i: (i, i * 0)),
        ],
        out_specs=pl.BlockSpec((_PE_BLOCK, OUT_COLS), lambda i: (i, i * 0)),
        out_shape=jax.ShapeDtypeStruct((n, OUT_COLS), jnp.float32),
    )(points, B_pe, gathered)


def kernel(points, values, mem, B_pe):
    del mem
    keys = _compute_keys(points).reshape((N_POINTS,))
    pkh, lh, lo = _sc_sort(keys)
    gout = _sc_accumulate(pkh, lh, lo, values)
    return _pe_concat(points, B_pe, gout)
